# Initial kernel scaffold; baseline (speedup 1.0000x reference)
import jax
import jax.numpy as jnp
from jax.experimental import pallas as pl


def kernel(x, codebook):
    raise NotImplementedError("write your pallas kernel here")


# speedup vs baseline: 1.6259x; 1.6259x over previous
import functools

import jax
import jax.numpy as jnp
from jax import lax
from jax.experimental import pallas as pl
from jax.experimental.pallas import tpu as pltpu

_K = 8192
_D = 32
_R = 256
_TEMP = 0.01
_COMMIT = 0.25
_ENT_RATIO = 0.1


def _vq_body(x_ref, cb_ref, idx_ref, q_ref, loss_ref, ap_acc, sc_acc):
    i = pl.program_id(0)
    nb = pl.num_programs(0)

    x = x_ref[...]
    cb = cb_ref[...]

    a2 = jnp.sum(x * x, axis=1, keepdims=True)
    b2 = jnp.sum(cb * cb, axis=1)[None, :]
    ab = lax.dot_general(x, cb, (((1,), (1,)), ((), ())),
                         preferred_element_type=jnp.float32)
    dist = a2 - 2.0 * ab + b2

    idx = jnp.argmin(dist, axis=-1).astype(jnp.int32)
    minval = jnp.min(dist, axis=-1)

    onehot = (lax.broadcasted_iota(jnp.int32, (_R, _K), 1)
              == idx[:, None]).astype(jnp.float32)
    q_ref[...] = lax.dot_general(onehot, cb, (((1,), (0,)), ((), ())),
                                 preferred_element_type=jnp.float32)

    logits = (0.0 - dist) / _TEMP
    m = jnp.max(logits, axis=-1, keepdims=True)
    e = jnp.exp(logits - m)
    s = jnp.sum(e, axis=-1)
    t = jnp.sum(e * (logits - m), axis=-1)
    se_contrib = jnp.sum(jnp.log(s) - t / s)
    colsum = jnp.sum(e / s[:, None], axis=0, keepdims=True)
    mse_contrib = jnp.sum(minval)

    idx_ref[...] = idx[None, None, :]

    @pl.when(i == 0)
    def _init():
        ap_acc[...] = jnp.zeros_like(ap_acc)
        sc_acc[0] = 0.0
        sc_acc[1] = 0.0

    ap_acc[...] += colsum
    sc_acc[0] += se_contrib
    sc_acc[1] += mse_contrib

    @pl.when(i == nb - 1)
    def _finish():
        n_tokens = nb * _R
        ap = ap_acc[...] / n_tokens
        avg_entropy = -jnp.sum(ap * jnp.log(ap + 1e-5))
        sample_entropy = sc_acc[0] / n_tokens
        mse_mean = sc_acc[1] / (n_tokens * _D)
        entropy_loss = (sample_entropy - avg_entropy) * _ENT_RATIO
        loss_ref[0, 0] = (1.0 + _COMMIT) * mse_mean + entropy_loss


@functools.partial(jax.jit, static_argnames=("interpret",))
def _vq_call(x2d, codebook, interpret=False):
    n = x2d.shape[0]
    nb = n // _R
    idx3d, q2d, loss = pl.pallas_call(
        _vq_body,
        grid=(nb,),
        in_specs=[
            pl.BlockSpec((_R, _D), lambda i: (i, 0)),
            pl.BlockSpec((_K, _D), lambda i: (0, 0)),
        ],
        out_specs=[
            pl.BlockSpec((1, 1, _R), lambda i: (i, 0, 0)),
            pl.BlockSpec((_R, _D), lambda i: (i, 0)),
            pl.BlockSpec(memory_space=pltpu.SMEM),
        ],
        out_shape=[
            jax.ShapeDtypeStruct((nb, 1, _R), jnp.int32),
            jax.ShapeDtypeStruct((n, _D), jnp.float32),
            jax.ShapeDtypeStruct((1, 1), jnp.float32),
        ],
        scratch_shapes=[
            pltpu.VMEM((1, _K), jnp.float32),
            pltpu.SMEM((2,), jnp.float32),
        ],
        interpret=interpret,
    )(x2d, codebook)
    return idx3d, q2d, loss


def kernel(x, codebook):
    codebook = jnp.asarray(codebook, dtype=jnp.float32)
    x2d = jnp.reshape(x, (-1, _D))
    idx3d, q2d, loss = _vq_call(x2d, codebook)
    encoding_indices = jnp.reshape(idx3d, x.shape[:-1])
    quantized = jnp.reshape(q2d, x.shape)
    return quantized, loss[0, 0], encoding_indices


# speedup vs baseline: 2.0006x; 1.2305x over previous
import functools

import jax
import jax.numpy as jnp
from jax import lax
from jax.experimental import pallas as pl
from jax.experimental.pallas import tpu as pltpu

_K = 8192
_D = 32
_R = 256
_TEMP = 0.01
_COMMIT = 0.25
_ENT_RATIO = 0.1


def _vq_body(x_ref, cb_ref, idx_ref, q_ref, loss_ref, b2_acc, ap_acc, sc_acc):
    i = pl.program_id(0)
    nb = pl.num_programs(0)

    x = x_ref[...]
    cb = cb_ref[...]

    @pl.when(i == 0)
    def _init():
        b2_acc[...] = jnp.sum(cb * cb, axis=1)[None, :]
        ap_acc[...] = jnp.zeros_like(ap_acc)
        sc_acc[0] = 0.0
        sc_acc[1] = 0.0

    a2 = jnp.sum(x * x, axis=1, keepdims=True)
    ab = lax.dot_general(x, cb, (((1,), (1,)), ((), ())),
                         preferred_element_type=jnp.float32)
    dist = a2 - 2.0 * ab + b2_acc[...]

    idx = jnp.argmin(dist, axis=-1).astype(jnp.int32)
    minval = jnp.min(dist, axis=-1)

    onehot = (lax.broadcasted_iota(jnp.int32, (_R, _K), 1)
              == idx[:, None]).astype(jnp.float32)
    q_ref[...] = lax.dot_general(onehot, cb, (((1,), (0,)), ((), ())),
                                 preferred_element_type=jnp.float32)

    u = (minval[:, None] - dist) * (1.0 / _TEMP)
    e = jnp.exp(u)
    s = jnp.sum(e, axis=-1)
    t = jnp.sum(e * u, axis=-1)
    rs = 1.0 / s
    se_contrib = jnp.sum(jnp.log(s) - t * rs)
    colsum = lax.dot_general(rs[None, :], e, (((1,), (0,)), ((), ())),
                             preferred_element_type=jnp.float32)
    mse_contrib = jnp.sum(minval)

    idx_ref[...] = idx[None, None, :]

    ap_acc[...] += colsum
    sc_acc[0] += se_contrib
    sc_acc[1] += mse_contrib

    @pl.when(i == nb - 1)
    def _finish():
        n_tokens = nb * _R
        ap = ap_acc[...] / n_tokens
        avg_entropy = -jnp.sum(ap * jnp.log(ap + 1e-5))
        sample_entropy = sc_acc[0] / n_tokens
        mse_mean = sc_acc[1] / (n_tokens * _D)
        entropy_loss = (sample_entropy - avg_entropy) * _ENT_RATIO
        loss_ref[0, 0] = (1.0 + _COMMIT) * mse_mean + entropy_loss


@functools.partial(jax.jit, static_argnames=("interpret",))
def _vq_call(x2d, codebook, interpret=False):
    n = x2d.shape[0]
    nb = n // _R
    idx3d, q2d, loss = pl.pallas_call(
        _vq_body,
        grid=(nb,),
        in_specs=[
            pl.BlockSpec((_R, _D), lambda i: (i, 0)),
            pl.BlockSpec((_K, _D), lambda i: (0, 0)),
        ],
        out_specs=[
            pl.BlockSpec((1, 1, _R), lambda i: (i, 0, 0)),
            pl.BlockSpec((_R, _D), lambda i: (i, 0)),
            pl.BlockSpec(memory_space=pltpu.SMEM),
        ],
        out_shape=[
            jax.ShapeDtypeStruct((nb, 1, _R), jnp.int32),
            jax.ShapeDtypeStruct((n, _D), jnp.float32),
            jax.ShapeDtypeStruct((1, 1), jnp.float32),
        ],
        scratch_shapes=[
            pltpu.VMEM((1, _K), jnp.float32),
            pltpu.VMEM((1, _K), jnp.float32),
            pltpu.SMEM((2,), jnp.float32),
        ],
        interpret=interpret,
    )(x2d, codebook)
    return idx3d, q2d, loss


def kernel(x, codebook):
    codebook = jnp.asarray(codebook, dtype=jnp.float32)
    x2d = jnp.reshape(x, (-1, _D))
    idx3d, q2d, loss = _vq_call(x2d, codebook)
    encoding_indices = jnp.reshape(idx3d, x.shape[:-1])
    quantized = jnp.reshape(q2d, x.shape)
    return quantized, loss[0, 0], encoding_indices


# speedup vs baseline: 2.3159x; 1.1576x over previous
import functools

import jax
import jax.numpy as jnp
from jax import lax
from jax.experimental import pallas as pl
from jax.experimental.pallas import tpu as pltpu
from jax.experimental.pallas import tpu_sc as plsc

_K = 8192
_D = 32
_R = 256
_TEMP = 0.01
_COMMIT = 0.25
_ENT_RATIO = 0.1


def _vq_body(x_ref, cb_ref, a2_ref, b2_ref, idx_ref, loss_ref, ap_acc, sc_acc):
    i = pl.program_id(0)
    nb = pl.num_programs(0)

    x = x_ref[...]
    cb = cb_ref[...]

    @pl.when(i == 0)
    def _init():
        ap_acc[...] = jnp.zeros_like(ap_acc)
        sc_acc[0] = 0.0
        sc_acc[1] = 0.0

    ab = lax.dot_general(x, cb, (((1,), (1,)), ((), ())),
                         preferred_element_type=jnp.float32)
    dist = a2_ref[...][:, :1] - 2.0 * ab + b2_ref[...]

    minval = jnp.min(dist, axis=-1)
    iota = lax.broadcasted_iota(jnp.int32, (_R, _K), 1)
    idx = jnp.min(jnp.where(dist == minval[:, None], iota, _K), axis=-1)

    u = (minval[:, None] - dist) * (1.0 / _TEMP)
    e = jnp.exp(u)
    s = jnp.sum(e, axis=-1)
    t = jnp.sum(e * u, axis=-1)
    rs = 1.0 / s
    se_contrib = jnp.sum(jnp.log(s) - t * rs)
    colsum = lax.dot_general(rs[None, :], e, (((1,), (0,)), ((), ())),
                             preferred_element_type=jnp.float32)
    mse_contrib = jnp.sum(minval)

    idx_ref[...] = idx[None, None, :]

    ap_acc[...] += colsum
    sc_acc[0] += se_contrib
    sc_acc[1] += mse_contrib

    @pl.when(i == nb - 1)
    def _finish():
        n_tokens = nb * _R
        ap = ap_acc[...] / n_tokens
        avg_entropy = -jnp.sum(ap * jnp.log(ap + 1e-5))
        sample_entropy = sc_acc[0] / n_tokens
        mse_mean = sc_acc[1] / (n_tokens * _D)
        entropy_loss = (sample_entropy - avg_entropy) * _ENT_RATIO
        loss_ref[0, 0] = (1.0 + _COMMIT) * mse_mean + entropy_loss


@functools.partial(jax.jit, static_argnames=("interpret",))
def _vq_call(x2d, codebook, a2b, b2, interpret=False):
    n = x2d.shape[0]
    nb = n // _R
    idx3d, loss = pl.pallas_call(
        _vq_body,
        grid=(nb,),
        in_specs=[
            pl.BlockSpec((_R, _D), lambda i: (i, 0)),
            pl.BlockSpec((_K, _D), lambda i: (0, 0)),
            pl.BlockSpec((_R, 128), lambda i: (i, 0)),
            pl.BlockSpec((1, _K), lambda i: (0, 0)),
        ],
        out_specs=[
            pl.BlockSpec((1, 1, _R), lambda i: (i, 0, 0)),
            pl.BlockSpec(memory_space=pltpu.SMEM),
        ],
        out_shape=[
            jax.ShapeDtypeStruct((nb, 1, _R), jnp.int32),
            jax.ShapeDtypeStruct((1, 1), jnp.float32),
        ],
        scratch_shapes=[
            pltpu.VMEM((1, _K), jnp.float32),
            pltpu.SMEM((2,), jnp.float32),
        ],
        interpret=interpret,
    )(x2d, codebook, a2b, b2)
    return idx3d, loss


_NC, _NS = 2, 16
_NW = _NC * _NS


_PADW = 128


def _sc_gather(table_pad, idx2d):
    b = idx2d.shape[0] * idx2d.shape[1]
    bpw = b // _NW
    nch = bpw // _PADW
    mesh = plsc.VectorSubcoreMesh(core_axis_name="c", subcore_axis_name="s")

    def _body(table_hbm, idx_hbm, out_hbm, idx_v, rows_v, sem):
        wid = lax.axis_index("s") * _NC + lax.axis_index("c")
        pltpu.sync_copy(idx_hbm.at[pl.ds(wid * nch, nch)], idx_v)
        copies = [
            pltpu.async_copy(table_hbm.at[idx_v.at[j]],
                             rows_v.at[pl.ds(j * _PADW, _PADW)], sem)
            for j in range(nch)
        ]
        for c in copies:
            c.wait()
        pltpu.sync_copy(rows_v, out_hbm.at[pl.ds(wid * bpw, bpw)])

    return pl.kernel(
        _body,
        out_type=jax.ShapeDtypeStruct((b, _PADW), jnp.float32),
        mesh=mesh,
        scratch_types=[
            pltpu.VMEM((nch, _PADW), jnp.int32),
            pltpu.VMEM((bpw, _PADW), jnp.float32),
            pltpu.SemaphoreType.DMA,
        ],
    )(table_pad, idx2d)


def kernel(x, codebook):
    codebook = jnp.asarray(codebook, dtype=jnp.float32)
    x2d = jnp.reshape(x, (-1, _D))
    a2 = jnp.sum(x2d ** 2, axis=1, keepdims=True)
    b2 = jnp.sum(codebook.T ** 2, axis=0, keepdims=True)
    a2b = jnp.broadcast_to(a2, (x2d.shape[0], 128))
    idx3d, loss = _vq_call(x2d, codebook, a2b, b2)
    table_pad = jnp.pad(codebook, ((0, 0), (0, _PADW - _D)))
    idx2d = jnp.reshape(idx3d, (-1, _PADW))
    qpad = _sc_gather(table_pad, idx2d)
    encoding_indices = jnp.reshape(idx3d, x.shape[:-1])
    quantized = jnp.reshape(qpad[:, :_D], x.shape)
    return quantized, loss[0, 0], encoding_indices


# speedup vs baseline: 2.4061x; 1.0390x over previous
import functools

import jax
import jax.numpy as jnp
from jax import lax
from jax.experimental import pallas as pl
from jax.experimental.pallas import tpu as pltpu
from jax.experimental.pallas import tpu_sc as plsc

_K = 8192
_D = 32
_R = 256
_TEMP = 0.01
_COMMIT = 0.25
_ENT_RATIO = 0.1


def _vq_body(x_ref, cb_ref, a2_ref, b2_ref, rev_ref, idx_ref, loss_ref,
             ap_acc, sc_acc):
    i = pl.program_id(0)
    nb = pl.num_programs(0)

    x = x_ref[...]
    cb = cb_ref[...]

    @pl.when(i == 0)
    def _init():
        ap_acc[...] = jnp.zeros_like(ap_acc)
        sc_acc[0] = 0.0
        sc_acc[1] = 0.0

    ab = lax.dot_general(x, cb, (((1,), (1,)), ((), ())),
                         preferred_element_type=jnp.float32)
    dist = a2_ref[...][:, :1] - 2.0 * ab + b2_ref[...]

    minval = jnp.min(dist, axis=-1)
    tieval = jnp.max(jnp.where(dist == minval[:, None], rev_ref[...], 0.0),
                     axis=-1)
    idx = _K - tieval.astype(jnp.int32)

    u = (minval[:, None] - dist) * (1.0 / _TEMP)
    e = jnp.exp(u)
    s = jnp.sum(e, axis=-1)
    t = jnp.sum(e * u, axis=-1)
    rs = 1.0 / s
    se_contrib = jnp.sum(jnp.log(s) - t * rs)
    colsum = lax.dot_general(rs[None, :], e, (((1,), (0,)), ((), ())),
                             preferred_element_type=jnp.float32)
    mse_contrib = jnp.sum(minval)

    idx_ref[...] = idx[None, None, :]

    ap_acc[...] += colsum
    sc_acc[0] += se_contrib
    sc_acc[1] += mse_contrib

    @pl.when(i == nb - 1)
    def _finish():
        n_tokens = nb * _R
        ap = ap_acc[...] / n_tokens
        avg_entropy = -jnp.sum(ap * jnp.log(ap + 1e-5))
        sample_entropy = sc_acc[0] / n_tokens
        mse_mean = sc_acc[1] / (n_tokens * _D)
        entropy_loss = (sample_entropy - avg_entropy) * _ENT_RATIO
        loss_ref[0, 0] = (1.0 + _COMMIT) * mse_mean + entropy_loss


@functools.partial(jax.jit, static_argnames=("interpret",))
def _vq_call(x2d, codebook, a2b, b2, interpret=False):
    n = x2d.shape[0]
    nb = n // _R
    idx3d, loss = pl.pallas_call(
        _vq_body,
        grid=(nb,),
        in_specs=[
            pl.BlockSpec((_R, _D), lambda i: (i, 0)),
            pl.BlockSpec((_K, _D), lambda i: (0, 0)),
            pl.BlockSpec((_R, 128), lambda i: (i, 0)),
            pl.BlockSpec((1, _K), lambda i: (0, 0)),
            pl.BlockSpec((1, _K), lambda i: (0, 0)),
        ],
        out_specs=[
            pl.BlockSpec((1, 1, _R), lambda i: (i, 0, 0)),
            pl.BlockSpec(memory_space=pltpu.SMEM),
        ],
        out_shape=[
            jax.ShapeDtypeStruct((nb, 1, _R), jnp.int32),
            jax.ShapeDtypeStruct((1, 1), jnp.float32),
        ],
        scratch_shapes=[
            pltpu.VMEM((1, _K), jnp.float32),
            pltpu.SMEM((2,), jnp.float32),
        ],
        interpret=interpret,
    )(x2d, codebook, a2b, b2,
      (_K - lax.iota(jnp.float32, _K))[None, :])
    return idx3d, loss


_NC, _NS = 2, 16
_NW = _NC * _NS


_PADW = 128


def _sc_gather(table_pad, idx2d):
    b = idx2d.shape[0] * idx2d.shape[1]
    bpw = b // _NW
    nch = bpw // _PADW
    mesh = plsc.VectorSubcoreMesh(core_axis_name="c", subcore_axis_name="s")

    def _body(table_hbm, idx_hbm, out_hbm, idx_v, rows_v, sem):
        wid = lax.axis_index("s") * _NC + lax.axis_index("c")
        pltpu.sync_copy(idx_hbm.at[pl.ds(wid * nch, nch)], idx_v)
        copies = [
            pltpu.async_copy(table_hbm.at[idx_v.at[j]],
                             rows_v.at[pl.ds(j * _PADW, _PADW)], sem)
            for j in range(nch)
        ]
        for c in copies:
            c.wait()
        pltpu.sync_copy(rows_v, out_hbm.at[pl.ds(wid * bpw, bpw)])

    return pl.kernel(
        _body,
        out_type=jax.ShapeDtypeStruct((b, _PADW), jnp.float32),
        mesh=mesh,
        scratch_types=[
            pltpu.VMEM((nch, _PADW), jnp.int32),
            pltpu.VMEM((bpw, _PADW), jnp.float32),
            pltpu.SemaphoreType.DMA,
        ],
    )(table_pad, idx2d)


def kernel(x, codebook):
    codebook = jnp.asarray(codebook, dtype=jnp.float32)
    x2d = jnp.reshape(x, (-1, _D))
    a2 = jnp.sum(x2d ** 2, axis=1, keepdims=True)
    b2 = jnp.sum(codebook.T ** 2, axis=0, keepdims=True)
    a2b = jnp.broadcast_to(a2, (x2d.shape[0], 128))
    idx3d, loss = _vq_call(x2d, codebook, a2b, b2)
    table_pad = jnp.pad(codebook, ((0, 0), (0, _PADW - _D)))
    idx2d = jnp.reshape(idx3d, (-1, _PADW))
    qpad = _sc_gather(table_pad, idx2d)
    encoding_indices = jnp.reshape(idx3d, x.shape[:-1])
    quantized = jnp.reshape(qpad[:, :_D], x.shape)
    return quantized, loss[0, 0], encoding_indices


# speedup vs baseline: 2.4145x; 1.0035x over previous
import functools

import jax
import jax.numpy as jnp
from jax import lax
from jax.experimental import pallas as pl
from jax.experimental.pallas import tpu as pltpu
from jax.experimental.pallas import tpu_sc as plsc

_K = 8192
_D = 32
_R = 256
_TEMP = 0.01
_COMMIT = 0.25
_ENT_RATIO = 0.1


def _vq_body(x_ref, cb_ref, a2_ref, b2_ref, rev_ref, idx_ref,
             loss_ref, ap_acc, sc_acc):
    i = pl.program_id(0)
    nb = pl.num_programs(0)

    x = x_ref[...]
    cb = cb_ref[...]

    @pl.when(i == 0)
    def _init():
        ap_acc[...] = jnp.zeros_like(ap_acc)
        sc_acc[0] = 0.0
        sc_acc[1] = 0.0

    ab = lax.dot_general(x, cb, (((1,), (1,)), ((), ())),
                         preferred_element_type=jnp.float32)
    dist = a2_ref[...][:, :1] - 2.0 * ab + b2_ref[...]

    minval = jnp.min(dist, axis=-1)

    u = (minval[:, None] - dist) * (1.0 / _TEMP)
    tieval = jnp.max(jnp.where(u == 0.0, rev_ref[...], 0.0), axis=-1)
    idx = _K - tieval.astype(jnp.int32)
    e = jnp.exp(u)
    s = jnp.sum(e, axis=-1)
    t = jnp.sum(e * u, axis=-1)
    rs = 1.0 / s
    se_contrib = jnp.sum(jnp.log(s) - t * rs)
    colsum = lax.dot_general(rs[None, :], e, (((1,), (0,)), ((), ())),
                             preferred_element_type=jnp.float32)
    mse_contrib = jnp.sum(minval)

    idx_ref[...] = idx[None, None, :]

    ap_acc[...] += colsum
    sc_acc[0] += se_contrib
    sc_acc[1] += mse_contrib

    @pl.when(i == nb - 1)
    def _finish():
        n_tokens = nb * _R
        ap = ap_acc[...] / n_tokens
        avg_entropy = -jnp.sum(ap * jnp.log(ap + 1e-5))
        sample_entropy = sc_acc[0] / n_tokens
        mse_mean = sc_acc[1] / (n_tokens * _D)
        entropy_loss = (sample_entropy - avg_entropy) * _ENT_RATIO
        loss_ref[0, 0] = (1.0 + _COMMIT) * mse_mean + entropy_loss


@functools.partial(jax.jit, static_argnames=("interpret",))
def _vq_call(x2d, codebook, a2b, b2, interpret=False):
    n = x2d.shape[0]
    nb = n // _R
    idx3d, loss = pl.pallas_call(
        _vq_body,
        grid=(nb,),
        in_specs=[
            pl.BlockSpec((_R, _D), lambda i: (i, 0)),
            pl.BlockSpec((_K, _D), lambda i: (0, 0)),
            pl.BlockSpec((_R, 128), lambda i: (i, 0)),
            pl.BlockSpec((1, _K), lambda i: (0, 0)),
            pl.BlockSpec((1, _K), lambda i: (0, 0)),
        ],
        out_specs=[
            pl.BlockSpec((1, 1, _R), lambda i: (i, 0, 0)),
            pl.BlockSpec(memory_space=pltpu.SMEM),
        ],
        out_shape=[
            jax.ShapeDtypeStruct((nb, 1, _R), jnp.int32),
            jax.ShapeDtypeStruct((1, 1), jnp.float32),
        ],
        scratch_shapes=[
            pltpu.VMEM((1, _K), jnp.float32),
            pltpu.SMEM((2,), jnp.float32),
        ],
        interpret=interpret,
    )(x2d, codebook, a2b, b2,
      (_K - lax.iota(jnp.float32, _K))[None, :])
    return idx3d, loss


_NC, _NS = 2, 16
_NW = _NC * _NS


_PADW = 128


def _sc_gather(table_pad, idx2d):
    b = idx2d.shape[0] * idx2d.shape[1]
    bpw = b // _NW
    nch = bpw // _PADW
    mesh = plsc.VectorSubcoreMesh(core_axis_name="c", subcore_axis_name="s")

    def _body(table_hbm, idx_hbm, out_hbm, idx_v, rows_v, sem):
        wid = lax.axis_index("s") * _NC + lax.axis_index("c")
        pltpu.sync_copy(idx_hbm.at[pl.ds(wid * nch, nch)], idx_v)
        copies = [
            pltpu.async_copy(table_hbm.at[idx_v.at[j]],
                             rows_v.at[pl.ds(j * _PADW, _PADW)], sem)
            for j in range(nch)
        ]
        for c in copies:
            c.wait()
        pltpu.sync_copy(rows_v, out_hbm.at[pl.ds(wid * bpw, bpw)])

    return pl.kernel(
        _body,
        out_type=jax.ShapeDtypeStruct((b, _PADW), jnp.float32),
        mesh=mesh,
        scratch_types=[
            pltpu.VMEM((nch, _PADW), jnp.int32),
            pltpu.VMEM((bpw, _PADW), jnp.float32),
            pltpu.SemaphoreType.DMA,
        ],
    )(table_pad, idx2d)


def kernel(x, codebook):
    codebook = jnp.asarray(codebook, dtype=jnp.float32)
    x2d = jnp.reshape(x, (-1, _D))
    a2 = jnp.sum(x2d ** 2, axis=1, keepdims=True)
    b2 = jnp.sum(codebook.T ** 2, axis=0, keepdims=True)
    a2b = jnp.broadcast_to(a2, (x2d.shape[0], 128))
    idx3d, loss = _vq_call(x2d, codebook, a2b, b2)
    table_pad = jnp.pad(codebook, ((0, 0), (0, _PADW - _D)))
    idx2d = jnp.reshape(idx3d, (-1, _PADW))
    qpad = _sc_gather(table_pad, idx2d)
    encoding_indices = jnp.reshape(idx3d, x.shape[:-1])
    quantized = jnp.reshape(qpad[:, :_D], x.shape)
    return quantized, loss[0, 0], encoding_indices
